# baseline (speedup 1.0000x reference)
import functools

import jax
import jax.numpy as jnp
from jax import lax
from jax.experimental import pallas as pl
from jax.experimental.pallas import tpu as pltpu
from jax.experimental.pallas import tpu_sc as plsc

_N = 10000
_E = 320000
_D = 128
_NW = 32
_EPT = _E // _NW
_EC = 40
_NCHUNK = _EPT // _EC
_ND = 1256
_F32 = jnp.float32


def _lrelu(v):
    return jnp.where(v >= 0.0, v, 0.2 * v)


def _head_bcast_mat(rows, heads):
    c = 128 // heads
    p = lax.broadcasted_iota(jnp.int32, (rows, 128), 0)
    j = lax.broadcasted_iota(jnp.int32, (rows, 128), 1) // c
    return (p == j).astype(_F32)


def _proj_and_shift(h, wlT, bl, wrT, br, attf, heads, xl_ref, xr_ref, sh_ref):
    xl = jnp.dot(h, wlT, preferred_element_type=_F32) + bl
    xr = jnp.dot(h, wrT, preferred_element_type=_F32) + br
    xl_ref[...] = xl
    xr_ref[...] = xr
    mx = jnp.max(xl, axis=0, keepdims=True) + jnp.max(xr, axis=0, keepdims=True)
    mn = jnp.min(xl, axis=0, keepdims=True) + jnp.min(xr, axis=0, keepdims=True)
    t = jnp.maximum(attf * _lrelu(mx), attf * _lrelu(mn))
    sh_ref[...] = jnp.dot(t, _head_bcast_mat(128, heads),
                          preferred_element_type=_F32)


def _tc_head_body(heads, x_ref, wlT_ref, bl_ref, wrT_ref, br_ref, attf_ref,
                  xl_ref, xr_ref, sh_ref):
    _proj_and_shift(x_ref[...], wlT_ref[...], bl_ref[...], wrT_ref[...],
                    br_ref[...], attf_ref[...], heads, xl_ref, xr_ref, sh_ref)


def _normalize(n0, n1, d0, d1, bias, heads_prev):
    num = n0 + n1
    den = jnp.dot(d0 + d1, _head_bcast_mat(16, heads_prev),
                  preferred_element_type=_F32)
    mask = den > 0.0
    g = jnp.where(mask, num / jnp.where(mask, den, 1.0), 0.0)
    return g + bias


def _tc_mid_body(heads_prev, heads, n0_ref, n1_ref, d0_ref, d1_ref, bias_ref,
                 wlT_ref, bl_ref, wrT_ref, br_ref, attf_ref,
                 xl_ref, xr_ref, sh_ref):
    g = _normalize(n0_ref[...], n1_ref[...], d0_ref[...], d1_ref[...],
                   bias_ref[...], heads_prev)
    h = jnp.where(g > 0.0, g, jnp.exp(g) - 1.0)
    _proj_and_shift(h, wlT_ref[...], bl_ref[...], wrT_ref[...], br_ref[...],
                    attf_ref[...], heads, xl_ref, xr_ref, sh_ref)


def _tc_out_body(heads_prev, n0_ref, n1_ref, d0_ref, d1_ref, bias_ref, out_ref):
    g = _normalize(n0_ref[...], n1_ref[...], d0_ref[...], d1_ref[...],
                   bias_ref[...], heads_prev)
    m = jnp.max(g, axis=1, keepdims=True)
    z = jnp.exp(g - m)
    out_ref[...] = z / jnp.sum(z, axis=1, keepdims=True)


def _tc_head(heads):
    return pl.pallas_call(
        functools.partial(_tc_head_body, heads),
        out_shape=[jax.ShapeDtypeStruct((_N, _D), _F32),
                   jax.ShapeDtypeStruct((_N, _D), _F32),
                   jax.ShapeDtypeStruct((1, _D), _F32)])


def _tc_mid(heads_prev, heads):
    return pl.pallas_call(
        functools.partial(_tc_mid_body, heads_prev, heads),
        out_shape=[jax.ShapeDtypeStruct((_N, _D), _F32),
                   jax.ShapeDtypeStruct((_N, _D), _F32),
                   jax.ShapeDtypeStruct((1, _D), _F32)])


def _tc_out(heads_prev):
    return pl.pallas_call(
        functools.partial(_tc_out_body, heads_prev),
        out_shape=jax.ShapeDtypeStruct((_N, _D), _F32))


def _sc_edge_body(heads, xl_hbm, xr_hbm, src_hbm, dst_hbm, dstp_hbm, att_hbm,
                  sh_hbm, num_hbm, den_hbm,
                  idx_sg, idx_dg, idx_ds, idx_dp, att_v, sh_v,
                  xjg, xi, alpha, contrib, denb, num_s, den_s,
                  semgx, semgi, semig, semis, semn, semd):
    c = 128 // heads
    k_per_head = c // 16
    cid = lax.axis_index("c")
    sid = lax.axis_index("s")
    wid = sid * 2 + cid
    ebase = wid * _EPT

    pltpu.sync_copy(att_hbm, att_v)
    pltpu.sync_copy(sh_hbm, sh_v)

    zero16 = jnp.zeros((16,), _F32)

    def _zero_row(e, carry):
        for k in range(8):
            contrib[e, pl.ds(k * 16, 16)] = zero16
        return carry

    lax.fori_loop(0, _EC, _zero_row, 0)

    def _for_row_chunks(nrows, step, fn):
        nchunks = nrows // step
        for r in range((nchunks + 15) // 16):
            ci = sid + 16 * r
            if (r + 1) * 16 <= nchunks:
                fn(pl.multiple_of(ci * step, step))
            else:
                @pl.when(ci < nchunks)
                def _():
                    fn(pl.multiple_of(ci * step, step))

    _for_row_chunks(_N, _EC, lambda rb: pltpu.sync_copy(
        contrib, num_s.at[pl.ds(rb, _EC)]))
    _for_row_chunks(_ND, 8, lambda rb: pltpu.sync_copy(
        contrib.at[pl.ds(0, 8)], den_s.at[pl.ds(rb, 8)]))
    plsc.subcore_barrier()

    lanes = lax.iota(jnp.int32, 16)

    def _sum_all_lanes(v):
        for k in (8, 4, 2, 1):
            v = v + v.at[lanes ^ k].get(mode="promise_in_bounds")
        return v

    one16 = jnp.ones((16,), _F32)
    lanesf = lanes.astype(_F32)
    head_onehot = [jnp.maximum(one16 - jnp.abs(lanesf - float(h)), 0.0)
                   for h in range(heads)]
    headmask = jnp.minimum(jnp.maximum(float(heads) - lanesf, 0.0), 1.0)
    shlane = zero16
    for h in range(heads):
        shlane = shlane + sh_v[0, pl.ds(h * c, 16)] * head_onehot[h]

    def _make_alpha(P):
        def _edge_alpha(e):
            arow = zero16
            for h in range(heads):
                acc = zero16
                for k in range(k_per_head):
                    off = h * c + k * 16
                    s_ = xjg[P][e, pl.ds(off, 16)] + xi[P][e, pl.ds(off, 16)]
                    acc = acc + _lrelu(s_) * att_v[0, pl.ds(off, 16)]
                arow = arow + _sum_all_lanes(acc) * head_onehot[h]
            alpha[e, pl.ds(0, 16)] = arow
        return _edge_alpha

    def _edge_exp(e):
        v = alpha[e, pl.ds(0, 16)]
        alpha[e, pl.ds(0, 16)] = jnp.exp(v - shlane) * headmask

    def _make_scale(P):
        def _edge_scale(e):
            exrow = alpha[e, pl.ds(0, 16)]
            for h in range(heads):
                exs = exrow.at[jnp.full((16,), h, jnp.int32)].get(
                    mode="promise_in_bounds")
                for k in range(k_per_head):
                    off = h * c + k * 16
                    contrib[e, pl.ds(off, 16)] = xjg[P][e, pl.ds(off, 16)] * exs
            start = jnp.minimum((e // 16) * 16, _EC - 16)
            dv = idx_dg[P][0, pl.ds(start, 16)]
            dsp = dv.at[jnp.full((16,), e - start, jnp.int32)].get(
                mode="promise_in_bounds")
            dmodf = jnp.bitwise_and(dsp, 7).astype(_F32)
            for b in range(8):
                ind = jnp.maximum(one16 - jnp.abs(dmodf - float(b)), 0.0)
                denb[e, pl.ds(b * 16, 16)] = exrow * ind
        return _edge_scale

    def _eb(j):
        return pl.multiple_of(ebase + j * _EC, 8)

    def _phase(j, P, first=False, prefetch=True):
        Q = 1 - P
        ebn = _eb(j + 1)
        ebj = _eb(j)
        pltpu.make_async_copy(xl_hbm.at[idx_sg[P].at[0]], xjg[P],
                              semgx[P]).wait()
        pltpu.make_async_copy(xr_hbm.at[idx_dg[P].at[0]], xi[P],
                              semgi[P]).wait()
        if prefetch:
            pltpu.async_copy(src_hbm.at[pl.ds(ebn, _EC)], idx_sg[Q].at[0],
                             semig[Q])
            pltpu.async_copy(dst_hbm.at[pl.ds(ebn, _EC)], idx_dg[Q].at[0],
                             semig[Q])
        plsc.parallel_loop(0, _EC, 1, unroll=5)(_make_alpha(P))
        if not first:
            pltpu.make_async_copy(contrib, num_s.at[idx_ds[Q].at[0]],
                                  semn[Q]).wait()
            pltpu.make_async_copy(denb, den_s.at[idx_dp[Q].at[0]],
                                  semd[Q]).wait()
        if prefetch:
            pltpu.async_copy(dst_hbm.at[pl.ds(ebn, _EC)], idx_ds[Q].at[0],
                             semis[Q])
            pltpu.async_copy(dstp_hbm.at[pl.ds(ebn, _EC)], idx_dp[Q].at[0],
                             semis[Q])
            pltpu.make_async_copy(src_hbm.at[pl.ds(ebn, _EC)],
                                  idx_sg[Q].at[0], semig[Q]).wait()
            pltpu.make_async_copy(dst_hbm.at[pl.ds(ebn, _EC)],
                                  idx_dg[Q].at[0], semig[Q]).wait()
            pltpu.async_copy(xl_hbm.at[idx_sg[Q].at[0]], xjg[Q], semgx[Q])
            pltpu.async_copy(xr_hbm.at[idx_dg[Q].at[0]], xi[Q], semgi[Q])
        plsc.parallel_loop(0, _EC, 1, unroll=8)(_edge_exp)
        plsc.parallel_loop(0, _EC, 1, unroll=5)(_make_scale(P))
        if not first:
            pltpu.make_async_copy(dst_hbm.at[pl.ds(ebj, _EC)],
                                  idx_ds[P].at[0], semis[P]).wait()
            pltpu.make_async_copy(dstp_hbm.at[pl.ds(ebj, _EC)],
                                  idx_dp[P].at[0], semis[P]).wait()
        pltpu.async_copy(contrib, num_s.at[idx_ds[P].at[0]], semn[P],
                         add=True)
        pltpu.async_copy(denb, den_s.at[idx_dp[P].at[0]], semd[P], add=True)

    eb0 = _eb(0)
    pltpu.sync_copy(src_hbm.at[pl.ds(eb0, _EC)], idx_sg[0].at[0])
    pltpu.sync_copy(dst_hbm.at[pl.ds(eb0, _EC)], idx_dg[0].at[0])
    pltpu.sync_copy(dst_hbm.at[pl.ds(eb0, _EC)], idx_ds[0].at[0])
    pltpu.sync_copy(dstp_hbm.at[pl.ds(eb0, _EC)], idx_dp[0].at[0])
    pltpu.async_copy(xl_hbm.at[idx_sg[0].at[0]], xjg[0], semgx[0])
    pltpu.async_copy(xr_hbm.at[idx_dg[0].at[0]], xi[0], semgi[0])

    _phase(0, 0, first=True)
    _phase(1, 1)

    def _pair(t, carry):
        _phase(2 * t, 0)
        _phase(2 * t + 1, 1)
        return carry

    lax.fori_loop(1, _NCHUNK // 2 - 1, _pair, 0)
    _phase(_NCHUNK - 2, 0)
    _phase(_NCHUNK - 1, 1, prefetch=False)
    pltpu.make_async_copy(contrib, num_s.at[idx_ds[1].at[0]], semn[1]).wait()
    pltpu.make_async_copy(denb, den_s.at[idx_dp[1].at[0]], semd[1]).wait()
    plsc.subcore_barrier()

    def _dump_num(rb):
        pltpu.sync_copy(num_s.at[pl.ds(rb, _EC)], contrib)
        pltpu.sync_copy(contrib, num_hbm.at[cid, pl.ds(rb, _EC)])

    def _dump_den(rb):
        pltpu.sync_copy(den_s.at[pl.ds(rb, 8)], contrib.at[pl.ds(0, 8)])
        pltpu.sync_copy(contrib.at[pl.ds(0, 8)], den_hbm.at[cid, pl.ds(rb, 8)])

    _for_row_chunks(_N, _EC, _dump_num)
    _for_row_chunks(_ND, 8, _dump_den)


def _sc_edge(heads):
    idx = pltpu.VMEM((1, _EC), jnp.int32)
    buf = pltpu.VMEM((_EC, _D), _F32)
    sem = pltpu.SemaphoreType.DMA
    return pl.kernel(
        functools.partial(_sc_edge_body, heads),
        out_type=[jax.ShapeDtypeStruct((2, _N, _D), _F32),
                  jax.ShapeDtypeStruct((2, _ND, _D), _F32)],
        scratch_types=[
            [idx, idx], [idx, idx], [idx, idx], [idx, idx],
            pltpu.VMEM((1, _D), _F32),
            pltpu.VMEM((1, _D), _F32),
            [buf, buf],
            [buf, buf],
            pltpu.VMEM((_EC, 16), _F32),
            buf,
            buf,
            pltpu.VMEM_SHARED((_N, _D), _F32),
            pltpu.VMEM_SHARED((_ND, _D), _F32),
            [sem, sem], [sem, sem], [sem, sem],
            [sem, sem], [sem, sem], [sem, sem],
        ],
        mesh=plsc.VectorSubcoreMesh(core_axis_name="c", subcore_axis_name="s"),
    )


def kernel(x, edge_index, Wl1, bl1, Wr1, br1, att1, b1,
           Wl2, bl2, Wr2, br2, att2, b2,
           Wl3, bl3, Wr3, br3, att3, b3):
    src1 = edge_index[0]
    dst1 = edge_index[1]
    dstp = jnp.right_shift(dst1, 3)

    def row(v):
        return v.reshape(1, _D)

    def run_layer(xl, xr, attf, sh, heads):
        num, den = _sc_edge(heads)(xl, xr, src1, dst1, dstp, attf, sh)
        d = den.reshape(2, _ND * 8, 16)[:, :_N, :]
        return num[0], num[1], d[0], d[1]

    xl, xr, sh = _tc_head(8)(x, Wl1.T, row(bl1), Wr1.T, row(br1),
                             att1.reshape(1, _D))
    n0, n1, d0, d1 = run_layer(xl, xr, att1.reshape(1, _D), sh, 8)

    xl, xr, sh = _tc_mid(8, 8)(n0, n1, d0, d1, row(b1), Wl2.T, row(bl2),
                               Wr2.T, row(br2), att2.reshape(1, _D))
    n0, n1, d0, d1 = run_layer(xl, xr, att2.reshape(1, _D), sh, 8)

    xl, xr, sh = _tc_mid(8, 1)(n0, n1, d0, d1, row(b2), Wl3.T, row(bl3),
                               Wr3.T, row(br3), att3.reshape(1, _D))
    n0, n1, d0, d1 = run_layer(xl, xr, att3.reshape(1, _D), sh, 1)

    return _tc_out(1)(n0, n1, d0, d1, row(b3))


# speedup vs baseline: 1.0981x; 1.0981x over previous
import functools

import jax
import jax.numpy as jnp
from jax import lax
from jax.experimental import pallas as pl
from jax.experimental.pallas import tpu as pltpu
from jax.experimental.pallas import tpu_sc as plsc

_N = 10000
_E = 320000
_D = 128
_NW = 32
_EPT = _E // _NW
_EC = 40
_NCHUNK = _EPT // _EC
_ND = 1256
_F32 = jnp.float32


def _lrelu(v):
    return jnp.where(v >= 0.0, v, 0.2 * v)


def _head_bcast_mat(rows, heads):
    c = 128 // heads
    p = lax.broadcasted_iota(jnp.int32, (rows, 128), 0)
    j = lax.broadcasted_iota(jnp.int32, (rows, 128), 1) // c
    return (p == j).astype(_F32)


def _proj_and_shift(h, wlT, bl, wrT, br, attf, heads, xl_ref, xr_ref, sh_ref):
    xl = jnp.dot(h, wlT, preferred_element_type=_F32) + bl
    xr = jnp.dot(h, wrT, preferred_element_type=_F32) + br
    xl_ref[...] = xl
    xr_ref[...] = xr
    mx = jnp.max(xl, axis=0, keepdims=True) + jnp.max(xr, axis=0, keepdims=True)
    mn = jnp.min(xl, axis=0, keepdims=True) + jnp.min(xr, axis=0, keepdims=True)
    t = jnp.maximum(attf * _lrelu(mx), attf * _lrelu(mn))
    sh_ref[...] = jnp.dot(t, _head_bcast_mat(128, heads),
                          preferred_element_type=_F32)


def _tc_head_body(heads, x_ref, wlT_ref, bl_ref, wrT_ref, br_ref, attf_ref,
                  xl_ref, xr_ref, sh_ref):
    _proj_and_shift(x_ref[...], wlT_ref[...], bl_ref[...], wrT_ref[...],
                    br_ref[...], attf_ref[...], heads, xl_ref, xr_ref, sh_ref)


def _normalize(n0, n1, d0, d1, bias, heads_prev):
    num = n0 + n1
    den = jnp.dot(d0 + d1, _head_bcast_mat(16, heads_prev),
                  preferred_element_type=_F32)
    mask = den > 0.0
    g = jnp.where(mask, num / jnp.where(mask, den, 1.0), 0.0)
    return g + bias


def _tc_mid_body(heads_prev, heads, n0_ref, n1_ref, d0_ref, d1_ref, bias_ref,
                 wlT_ref, bl_ref, wrT_ref, br_ref, attf_ref,
                 xl_ref, xr_ref, sh_ref):
    g = _normalize(n0_ref[...], n1_ref[...], d0_ref[...], d1_ref[...],
                   bias_ref[...], heads_prev)
    h = jnp.where(g > 0.0, g, jnp.exp(g) - 1.0)
    _proj_and_shift(h, wlT_ref[...], bl_ref[...], wrT_ref[...], br_ref[...],
                    attf_ref[...], heads, xl_ref, xr_ref, sh_ref)


def _tc_out_body(heads_prev, n0_ref, n1_ref, d0_ref, d1_ref, bias_ref, out_ref):
    g = _normalize(n0_ref[...], n1_ref[...], d0_ref[...], d1_ref[...],
                   bias_ref[...], heads_prev)
    m = jnp.max(g, axis=1, keepdims=True)
    z = jnp.exp(g - m)
    out_ref[...] = z / jnp.sum(z, axis=1, keepdims=True)


def _tc_head(heads):
    return pl.pallas_call(
        functools.partial(_tc_head_body, heads),
        out_shape=[jax.ShapeDtypeStruct((_N, _D), _F32),
                   jax.ShapeDtypeStruct((_N, _D), _F32),
                   jax.ShapeDtypeStruct((1, _D), _F32)])


def _tc_mid(heads_prev, heads):
    return pl.pallas_call(
        functools.partial(_tc_mid_body, heads_prev, heads),
        out_shape=[jax.ShapeDtypeStruct((_N, _D), _F32),
                   jax.ShapeDtypeStruct((_N, _D), _F32),
                   jax.ShapeDtypeStruct((1, _D), _F32)])


def _tc_out(heads_prev):
    return pl.pallas_call(
        functools.partial(_tc_out_body, heads_prev),
        out_shape=jax.ShapeDtypeStruct((_N, _D), _F32))


def _sc_edge_body(heads, xl_hbm, xr_hbm, src_hbm, dst_hbm, dstp_hbm, att_hbm,
                  sh_hbm, num_hbm, den_hbm,
                  idx_sg, idx_dg, idx_ds, idx_dp, att_v, sh_v,
                  xjg, xi, alpha, contrib, denb, num_s, den_s,
                  semgx, semgi, semig, semis, semn, semd):
    c = 128 // heads
    k_per_head = c // 16
    cid = lax.axis_index("c")
    sid = lax.axis_index("s")
    wid = sid * 2 + cid
    ebase = wid * _EPT

    pltpu.sync_copy(att_hbm, att_v)
    pltpu.sync_copy(sh_hbm, sh_v)

    zero16 = jnp.zeros((16,), _F32)

    def _zero_row(e, carry):
        for k in range(8):
            contrib[e, pl.ds(k * 16, 16)] = zero16
        return carry

    lax.fori_loop(0, _EC, _zero_row, 0)

    def _for_row_chunks(nrows, step, fn):
        nchunks = nrows // step
        for r in range((nchunks + 15) // 16):
            ci = sid + 16 * r
            if (r + 1) * 16 <= nchunks:
                fn(pl.multiple_of(ci * step, step))
            else:
                @pl.when(ci < nchunks)
                def _():
                    fn(pl.multiple_of(ci * step, step))

    _for_row_chunks(_N, _EC, lambda rb: pltpu.sync_copy(
        contrib, num_s.at[pl.ds(rb, _EC)]))
    _for_row_chunks(_ND, 8, lambda rb: pltpu.sync_copy(
        contrib.at[pl.ds(0, 8)], den_s.at[pl.ds(rb, 8)]))
    plsc.subcore_barrier()

    lanes = lax.iota(jnp.int32, 16)

    def _sum_all_lanes(v):
        for k in (8, 4, 2, 1):
            v = v + v.at[lanes ^ k].get(mode="promise_in_bounds")
        return v

    one16 = jnp.ones((16,), _F32)
    lanesf = lanes.astype(_F32)
    head_onehot = [jnp.maximum(one16 - jnp.abs(lanesf - float(h)), 0.0)
                   for h in range(heads)]
    headmask = jnp.minimum(jnp.maximum(float(heads) - lanesf, 0.0), 1.0)
    shlane = zero16
    for h in range(heads):
        shlane = shlane + sh_v[0, pl.ds(h * c, 16)] * head_onehot[h]

    def _make_alpha(P):
        def _edge_alpha(e):
            arow = zero16
            for h in range(heads):
                acc = zero16
                for k in range(k_per_head):
                    off = h * c + k * 16
                    s_ = xjg[P][e, pl.ds(off, 16)] + xi[P][e, pl.ds(off, 16)]
                    acc = acc + _lrelu(s_) * att_v[0, pl.ds(off, 16)]
                arow = arow + _sum_all_lanes(acc) * head_onehot[h]
            alpha[e, pl.ds(0, 16)] = arow
        return _edge_alpha

    def _edge_exp(e):
        v = alpha[e, pl.ds(0, 16)]
        alpha[e, pl.ds(0, 16)] = jnp.exp(v - shlane) * headmask

    def _make_scale(P):
        def _edge_scale(e):
            exrow = alpha[e, pl.ds(0, 16)]
            for h in range(heads):
                exs = exrow.at[jnp.full((16,), h, jnp.int32)].get(
                    mode="promise_in_bounds")
                for k in range(k_per_head):
                    off = h * c + k * 16
                    contrib[e, pl.ds(off, 16)] = xjg[P][e, pl.ds(off, 16)] * exs
            start = jnp.minimum((e // 16) * 16, _EC - 16)
            dv = idx_dg[P][0, pl.ds(start, 16)]
            dsp = dv.at[jnp.full((16,), e - start, jnp.int32)].get(
                mode="promise_in_bounds")
            dmodf = jnp.bitwise_and(dsp, 7).astype(_F32)
            for b in range(8):
                ind = jnp.maximum(one16 - jnp.abs(dmodf - float(b)), 0.0)
                denb[e, pl.ds(b * 16, 16)] = exrow * ind
        return _edge_scale

    def _eb(j):
        return pl.multiple_of(ebase + j * _EC, 8)

    def _phase(j, P, first=False, prefetch=True):
        Q = 1 - P
        ebn = _eb(j + 1)
        ebj = _eb(j)
        pltpu.make_async_copy(xl_hbm.at[idx_sg[P].at[0]], xjg[P],
                              semgx[P]).wait()
        pltpu.make_async_copy(xr_hbm.at[idx_dg[P].at[0]], xi[P],
                              semgi[P]).wait()
        if prefetch:
            pltpu.async_copy(src_hbm.at[pl.ds(ebn, _EC)], idx_sg[Q].at[0],
                             semig[Q])
            pltpu.async_copy(dst_hbm.at[pl.ds(ebn, _EC)], idx_dg[Q].at[0],
                             semig[Q])
        plsc.parallel_loop(0, _EC, 1, unroll=5)(_make_alpha(P))
        if not first:
            pltpu.make_async_copy(contrib, num_s.at[idx_ds[Q].at[0]],
                                  semn[Q]).wait()
            pltpu.make_async_copy(denb, den_s.at[idx_dp[Q].at[0]],
                                  semd[Q]).wait()
        if prefetch:
            pltpu.async_copy(dst_hbm.at[pl.ds(ebn, _EC)], idx_ds[Q].at[0],
                             semis[Q])
            pltpu.async_copy(dstp_hbm.at[pl.ds(ebn, _EC)], idx_dp[Q].at[0],
                             semis[Q])
            pltpu.make_async_copy(src_hbm.at[pl.ds(ebn, _EC)],
                                  idx_sg[Q].at[0], semig[Q]).wait()
            pltpu.make_async_copy(dst_hbm.at[pl.ds(ebn, _EC)],
                                  idx_dg[Q].at[0], semig[Q]).wait()
            pltpu.async_copy(xl_hbm.at[idx_sg[Q].at[0]], xjg[Q], semgx[Q])
            pltpu.async_copy(xr_hbm.at[idx_dg[Q].at[0]], xi[Q], semgi[Q])
        plsc.parallel_loop(0, _EC, 1, unroll=8)(_edge_exp)
        plsc.parallel_loop(0, _EC, 1, unroll=4)(_make_scale(P))
        if not first:
            pltpu.make_async_copy(dst_hbm.at[pl.ds(ebj, _EC)],
                                  idx_ds[P].at[0], semis[P]).wait()
            pltpu.make_async_copy(dstp_hbm.at[pl.ds(ebj, _EC)],
                                  idx_dp[P].at[0], semis[P]).wait()
        pltpu.async_copy(contrib, num_s.at[idx_ds[P].at[0]], semn[P],
                         add=True)
        pltpu.async_copy(denb, den_s.at[idx_dp[P].at[0]], semd[P], add=True)

    eb0 = _eb(0)
    pltpu.sync_copy(src_hbm.at[pl.ds(eb0, _EC)], idx_sg[0].at[0])
    pltpu.sync_copy(dst_hbm.at[pl.ds(eb0, _EC)], idx_dg[0].at[0])
    pltpu.sync_copy(dst_hbm.at[pl.ds(eb0, _EC)], idx_ds[0].at[0])
    pltpu.sync_copy(dstp_hbm.at[pl.ds(eb0, _EC)], idx_dp[0].at[0])
    pltpu.async_copy(xl_hbm.at[idx_sg[0].at[0]], xjg[0], semgx[0])
    pltpu.async_copy(xr_hbm.at[idx_dg[0].at[0]], xi[0], semgi[0])

    _phase(0, 0, first=True)
    _phase(1, 1)

    def _pair(t, carry):
        _phase(2 * t, 0)
        _phase(2 * t + 1, 1)
        return carry

    lax.fori_loop(1, _NCHUNK // 2 - 1, _pair, 0)
    _phase(_NCHUNK - 2, 0)
    _phase(_NCHUNK - 1, 1, prefetch=False)
    pltpu.make_async_copy(contrib, num_s.at[idx_ds[1].at[0]], semn[1]).wait()
    pltpu.make_async_copy(denb, den_s.at[idx_dp[1].at[0]], semd[1]).wait()
    plsc.subcore_barrier()

    def _dump_num(rb):
        pltpu.sync_copy(num_s.at[pl.ds(rb, _EC)], contrib)
        pltpu.sync_copy(contrib, num_hbm.at[cid, pl.ds(rb, _EC)])

    def _dump_den(rb):
        pltpu.sync_copy(den_s.at[pl.ds(rb, 8)], contrib.at[pl.ds(0, 8)])
        pltpu.sync_copy(contrib.at[pl.ds(0, 8)], den_hbm.at[cid, pl.ds(rb, 8)])

    _for_row_chunks(_N, _EC, _dump_num)
    _for_row_chunks(_ND, 8, _dump_den)


def _sc_edge(heads):
    idx = pltpu.VMEM((1, _EC), jnp.int32)
    buf = pltpu.VMEM((_EC, _D), _F32)
    sem = pltpu.SemaphoreType.DMA
    return pl.kernel(
        functools.partial(_sc_edge_body, heads),
        out_type=[jax.ShapeDtypeStruct((2, _N, _D), _F32),
                  jax.ShapeDtypeStruct((2, _ND, _D), _F32)],
        scratch_types=[
            [idx, idx], [idx, idx], [idx, idx], [idx, idx],
            pltpu.VMEM((1, _D), _F32),
            pltpu.VMEM((1, _D), _F32),
            [buf, buf],
            [buf, buf],
            pltpu.VMEM((_EC, 16), _F32),
            buf,
            buf,
            pltpu.VMEM_SHARED((_N, _D), _F32),
            pltpu.VMEM_SHARED((_ND, _D), _F32),
            [sem, sem], [sem, sem], [sem, sem],
            [sem, sem], [sem, sem], [sem, sem],
        ],
        mesh=plsc.VectorSubcoreMesh(core_axis_name="c", subcore_axis_name="s"),
    )


def kernel(x, edge_index, Wl1, bl1, Wr1, br1, att1, b1,
           Wl2, bl2, Wr2, br2, att2, b2,
           Wl3, bl3, Wr3, br3, att3, b3):
    src1 = edge_index[0]
    dst1 = edge_index[1]
    dstp = jnp.right_shift(dst1, 3)

    def row(v):
        return v.reshape(1, _D)

    def run_layer(xl, xr, attf, sh, heads):
        num, den = _sc_edge(heads)(xl, xr, src1, dst1, dstp, attf, sh)
        d = den.reshape(2, _ND * 8, 16)[:, :_N, :]
        return num[0], num[1], d[0], d[1]

    xl, xr, sh = _tc_head(8)(x, Wl1.T, row(bl1), Wr1.T, row(br1),
                             att1.reshape(1, _D))
    n0, n1, d0, d1 = run_layer(xl, xr, att1.reshape(1, _D), sh, 8)

    xl, xr, sh = _tc_mid(8, 8)(n0, n1, d0, d1, row(b1), Wl2.T, row(bl2),
                               Wr2.T, row(br2), att2.reshape(1, _D))
    n0, n1, d0, d1 = run_layer(xl, xr, att2.reshape(1, _D), sh, 8)

    xl, xr, sh = _tc_mid(8, 1)(n0, n1, d0, d1, row(b2), Wl3.T, row(bl3),
                               Wr3.T, row(br3), att3.reshape(1, _D))
    n0, n1, d0, d1 = run_layer(xl, xr, att3.reshape(1, _D), sh, 1)

    return _tc_out(1)(n0, n1, d0, d1, row(b3))
